# baseline (speedup 1.0000x reference)
import jax
import jax.numpy as jnp
from jax.experimental import pallas as pl
from jax.experimental.pallas import tpu as pltpu

N = 4096
NFEAT = 128
NHID = 128
BLK = 256
NBLK = N // BLK


def _prep_body(feat_ref, wft_ref, bf_ref, a2_ref, w0t_ref, b0_ref,
               f_ref, x0_ref):
    feat = feat_ref[...]
    h = jnp.dot(feat, wft_ref[...], preferred_element_type=jnp.float32)
    h = h + bf_ref[...]
    f_ref[...] = jnp.dot(h, a2_ref[...], preferred_element_type=jnp.float32)
    x0 = jnp.dot(feat, w0t_ref[...], preferred_element_type=jnp.float32)
    x0_ref[...] = x0 + b0_ref[...]


def _attn_body(f_ref, f2r_ref, x0_ref, w1t_ref, b1_ref, o_ref, y_scr):
    i = pl.program_id(0)
    base = (i % NBLK) * BLK
    c1 = jnp.max(f_ref[:, 0])
    c2 = jnp.max(f_ref[:, 1])
    f1 = f_ref[pl.ds(base, BLK), 0:1]
    f2 = f2r_ref[...]
    r = jnp.exp(0.8 * (f1 - c1))
    b1v = jnp.exp(f2 - c2)
    b2v = jnp.exp(0.2 * (f2 - c2) - 0.8 * (c1 + c2))
    e = jnp.maximum(r * b1v, b2v)
    s = jnp.sum(e, axis=1, keepdims=True)
    e = e.astype(jnp.bfloat16)

    @pl.when(i < NBLK)
    def _phase1():
        x = x0_ref[...].astype(jnp.bfloat16)
        p = jnp.dot(e, x, preferred_element_type=jnp.float32)
        y = p / s
        y = jnp.where(y > 0.0, y, jnp.exp(y) - 1.0)
        y_scr[pl.ds(base, BLK), :] = y
        o_ref[...] = y

    @pl.when(i >= NBLK)
    def _phase2():
        x = y_scr[...].astype(jnp.bfloat16)
        p = jnp.dot(e, x, preferred_element_type=jnp.float32)
        y = p / s
        z = jnp.dot(y, w1t_ref[...], preferred_element_type=jnp.float32)
        z = z + b1_ref[...]
        o_ref[...] = jnp.where(z > 0.0, z, jnp.exp(z) - 1.0)


@jax.jit
def kernel(feat_data, adjs, Wf, bf, a_src, a_dest, W0, b0, W1, b1):
    del adjs
    a2 = jnp.concatenate([a_src, a_dest], axis=1)

    f, x0 = pl.pallas_call(
        _prep_body,
        grid=(NBLK,),
        in_specs=[
            pl.BlockSpec((BLK, NFEAT), lambda i: (i, 0)),
            pl.BlockSpec((NFEAT, NHID), lambda i: (0, 0)),
            pl.BlockSpec((1, NHID), lambda i: (0, 0)),
            pl.BlockSpec((NHID, 2), lambda i: (0, 0)),
            pl.BlockSpec((NFEAT, NHID), lambda i: (0, 0)),
            pl.BlockSpec((1, NHID), lambda i: (0, 0)),
        ],
        out_specs=[
            pl.BlockSpec((BLK, 2), lambda i: (i, 0)),
            pl.BlockSpec((BLK, NHID), lambda i: (i, 0)),
        ],
        out_shape=[
            jax.ShapeDtypeStruct((N, 2), jnp.float32),
            jax.ShapeDtypeStruct((N, NHID), jnp.float32),
        ],
    )(feat_data, Wf.T, bf.reshape(1, NHID), a2, W0.T, b0.reshape(1, NHID))

    f2r = f[:, 1].reshape(1, N)

    out = pl.pallas_call(
        _attn_body,
        grid=(2 * NBLK,),
        in_specs=[
            pl.BlockSpec((N, 2), lambda i: (0, 0)),
            pl.BlockSpec((1, N), lambda i: (0, 0)),
            pl.BlockSpec((N, NHID), lambda i: (0, 0)),
            pl.BlockSpec((NHID, NHID), lambda i: (0, 0)),
            pl.BlockSpec((1, NHID), lambda i: (0, 0)),
        ],
        out_specs=pl.BlockSpec((BLK, NHID), lambda i: (i % NBLK, 0)),
        out_shape=jax.ShapeDtypeStruct((N, NHID), jnp.float32),
        scratch_shapes=[pltpu.VMEM((N, NHID), jnp.float32)],
        compiler_params=pltpu.CompilerParams(
            dimension_semantics=("arbitrary",)),
    )(f, f2r, x0, W1.T, b1.reshape(1, NHID))

    return out


# speedup vs baseline: 1.4917x; 1.4917x over previous
import jax
import jax.numpy as jnp
from jax.experimental import pallas as pl
from jax.experimental.pallas import tpu as pltpu

N = 4096
NFEAT = 128
NHID = 128
BLK = 256
NBLK = N // BLK


def _prep_body(feat_ref, wft_ref, bf_ref, a2_ref, w0t_ref, b0_ref,
               f_ref, x0_ref):
    feat = feat_ref[...]
    h = jnp.dot(feat, wft_ref[...], preferred_element_type=jnp.float32)
    h = h + bf_ref[...]
    f_ref[...] = jnp.dot(h, a2_ref[...], preferred_element_type=jnp.float32)
    x0 = jnp.dot(feat, w0t_ref[...], preferred_element_type=jnp.float32)
    x0_ref[...] = (x0 + b0_ref[...]).astype(jnp.bfloat16)


def _attn_body(f1_ref, f2r_ref, c_ref, x0_ref, w1t_ref, b1_ref,
               o_ref, y_scr):
    i = pl.program_id(0)
    c1 = c_ref[0, 0]
    c2 = c_ref[0, 1]
    f1 = f1_ref[...]
    f2 = f2r_ref[...]
    r = jnp.exp(0.8 * (f1 - c1))
    b1v = jnp.exp(f2 - c2)
    b2v = jnp.exp(0.2 * (f2 - c2) - 0.8 * (c1 + c2))
    e = jnp.maximum(r * b1v, b2v)
    s = jnp.sum(e, axis=1, keepdims=True)
    e = e.astype(jnp.bfloat16)
    base = (i % NBLK) * BLK

    @pl.when(i < NBLK)
    def _phase1():
        p = jnp.dot(e, x0_ref[...], preferred_element_type=jnp.float32)
        y = p / s
        y = jnp.where(y > 0.0, y, jnp.exp(y) - 1.0)
        y_scr[pl.ds(base, BLK), :] = y.astype(jnp.bfloat16)

    @pl.when(i >= NBLK)
    def _phase2():
        p = jnp.dot(e, y_scr[...], preferred_element_type=jnp.float32)
        y = p / s
        z = jnp.dot(y, w1t_ref[...], preferred_element_type=jnp.float32)
        z = z + b1_ref[...]
        o_ref[...] = jnp.where(z > 0.0, z, jnp.exp(z) - 1.0)


@jax.jit
def kernel(feat_data, adjs, Wf, bf, a_src, a_dest, W0, b0, W1, b1):
    del adjs
    a2 = jnp.concatenate([a_src, a_dest], axis=1)

    f, x0 = pl.pallas_call(
        _prep_body,
        grid=(NBLK,),
        in_specs=[
            pl.BlockSpec((BLK, NFEAT), lambda i: (i, 0)),
            pl.BlockSpec((NFEAT, NHID), lambda i: (0, 0)),
            pl.BlockSpec((1, NHID), lambda i: (0, 0)),
            pl.BlockSpec((NHID, 2), lambda i: (0, 0)),
            pl.BlockSpec((NFEAT, NHID), lambda i: (0, 0)),
            pl.BlockSpec((1, NHID), lambda i: (0, 0)),
        ],
        out_specs=[
            pl.BlockSpec((BLK, 2), lambda i: (i, 0)),
            pl.BlockSpec((BLK, NHID), lambda i: (i, 0)),
        ],
        out_shape=[
            jax.ShapeDtypeStruct((N, 2), jnp.float32),
            jax.ShapeDtypeStruct((N, NHID), jnp.bfloat16),
        ],
    )(feat_data, Wf.T, bf.reshape(1, NHID), a2, W0.T, b0.reshape(1, NHID))

    c = jnp.max(f, axis=0).reshape(1, 2)
    f1c = f[:, 0:1]
    f2r = f[:, 1].reshape(1, N)

    out = pl.pallas_call(
        _attn_body,
        grid=(2 * NBLK,),
        in_specs=[
            pl.BlockSpec((BLK, 1), lambda i: (i % NBLK, 0)),
            pl.BlockSpec((1, N), lambda i: (0, 0)),
            pl.BlockSpec((1, 2), lambda i: (0, 0)),
            pl.BlockSpec((N, NHID), lambda i: (0, 0)),
            pl.BlockSpec((NHID, NHID), lambda i: (0, 0)),
            pl.BlockSpec((1, NHID), lambda i: (0, 0)),
        ],
        out_specs=pl.BlockSpec(
            (BLK, NHID),
            lambda i: (jnp.where(i < NBLK, 0, i - NBLK), 0)),
        out_shape=jax.ShapeDtypeStruct((N, NHID), jnp.float32),
        scratch_shapes=[pltpu.VMEM((N, NHID), jnp.bfloat16)],
        compiler_params=pltpu.CompilerParams(
            dimension_semantics=("arbitrary",)),
    )(f1c, f2r, c, x0, W1.T, b1.reshape(1, NHID))

    return out


# speedup vs baseline: 1.6527x; 1.1080x over previous
import jax
import jax.numpy as jnp
from jax.experimental import pallas as pl

N = 4096
NFEAT = 128
NHID = 128
BLK = 256
NBLK = N // BLK


def _prep_body(feat_ref, wft_ref, bf_ref, a2_ref, w0t_ref, b0_ref,
               f_ref, x0_ref):
    feat = feat_ref[...]
    h = jnp.dot(feat, wft_ref[...], preferred_element_type=jnp.float32)
    h = h + bf_ref[...]
    f_ref[...] = jnp.dot(h, a2_ref[...], preferred_element_type=jnp.float32)
    x0 = jnp.dot(feat, w0t_ref[...], preferred_element_type=jnp.float32)
    x0_ref[...] = (x0 + b0_ref[...]).astype(jnp.bfloat16)


def _attn_factors(f1, f2, c1, c2):
    r = jnp.exp(0.8 * (f1 - c1))
    b1v = jnp.exp(f2 - c2)
    b2v = jnp.exp(0.2 * (f2 - c2) - 0.8 * (c1 + c2))
    e = jnp.maximum(r * b1v, b2v)
    s = jnp.sum(e, axis=1, keepdims=True)
    return e.astype(jnp.bfloat16), s


def _attn1_body(f1_ref, f2_ref, c_ref, x_ref, o_ref):
    e, s = _attn_factors(f1_ref[...], f2_ref[...], c_ref[0, 0], c_ref[0, 1])
    p = jnp.dot(e, x_ref[...], preferred_element_type=jnp.float32)
    y = p / s
    y = jnp.where(y > 0.0, y, jnp.exp(y) - 1.0)
    o_ref[...] = y.astype(jnp.bfloat16)


def _attn2_body(f1_ref, f2_ref, c_ref, x_ref, w1t_ref, b1_ref, o_ref):
    e, s = _attn_factors(f1_ref[...], f2_ref[...], c_ref[0, 0], c_ref[0, 1])
    p = jnp.dot(e, x_ref[...], preferred_element_type=jnp.float32)
    y = p / s
    z = jnp.dot(y, w1t_ref[...], preferred_element_type=jnp.float32)
    z = z + b1_ref[...]
    o_ref[...] = jnp.where(z > 0.0, z, jnp.exp(z) - 1.0)


@jax.jit
def kernel(feat_data, adjs, Wf, bf, a_src, a_dest, W0, b0, W1, b1):
    del adjs
    a2 = jnp.concatenate([a_src, a_dest], axis=1)

    f, x0 = pl.pallas_call(
        _prep_body,
        grid=(NBLK,),
        in_specs=[
            pl.BlockSpec((BLK, NFEAT), lambda i: (i, 0)),
            pl.BlockSpec((NFEAT, NHID), lambda i: (0, 0)),
            pl.BlockSpec((1, NHID), lambda i: (0, 0)),
            pl.BlockSpec((NHID, 2), lambda i: (0, 0)),
            pl.BlockSpec((NFEAT, NHID), lambda i: (0, 0)),
            pl.BlockSpec((1, NHID), lambda i: (0, 0)),
        ],
        out_specs=[
            pl.BlockSpec((BLK, 2), lambda i: (i, 0)),
            pl.BlockSpec((BLK, NHID), lambda i: (i, 0)),
        ],
        out_shape=[
            jax.ShapeDtypeStruct((N, 2), jnp.float32),
            jax.ShapeDtypeStruct((N, NHID), jnp.bfloat16),
        ],
    )(feat_data, Wf.T, bf.reshape(1, NHID), a2, W0.T, b0.reshape(1, NHID))

    c = jnp.max(f, axis=0).reshape(1, 2)
    f1c = f[:, 0:1]
    f2r = f[:, 1].reshape(1, N)

    attn_specs = [
        pl.BlockSpec((BLK, 1), lambda i: (i, 0)),
        pl.BlockSpec((1, N), lambda i: (0, 0)),
        pl.BlockSpec((1, 2), lambda i: (0, 0)),
        pl.BlockSpec((N, NHID), lambda i: (0, 0)),
    ]

    y0 = pl.pallas_call(
        _attn1_body,
        grid=(NBLK,),
        in_specs=attn_specs,
        out_specs=pl.BlockSpec((BLK, NHID), lambda i: (i, 0)),
        out_shape=jax.ShapeDtypeStruct((N, NHID), jnp.bfloat16),
    )(f1c, f2r, c, x0)

    out = pl.pallas_call(
        _attn2_body,
        grid=(NBLK,),
        in_specs=attn_specs + [
            pl.BlockSpec((NHID, NHID), lambda i: (0, 0)),
            pl.BlockSpec((1, NHID), lambda i: (0, 0)),
        ],
        out_specs=pl.BlockSpec((BLK, NHID), lambda i: (i, 0)),
        out_shape=jax.ShapeDtypeStruct((N, NHID), jnp.float32),
    )(f1c, f2r, c, y0, W1.T, b1.reshape(1, NHID))

    return out


# speedup vs baseline: 1.9649x; 1.1889x over previous
import jax
import jax.numpy as jnp
from jax.experimental import pallas as pl

N = 4096
NFEAT = 128
NHID = 128
BLK = 512
NBLK = N // BLK


def _prep_body(feat_ref, wft_ref, bf_ref, a2_ref, w0t_ref, b0_ref,
               f_ref, x0_ref):
    feat = feat_ref[...]
    h = jnp.dot(feat, wft_ref[...], preferred_element_type=jnp.float32)
    h = h + bf_ref[...]
    f_ref[...] = jnp.dot(h, a2_ref[...], preferred_element_type=jnp.float32)
    x0 = jnp.dot(feat, w0t_ref[...], preferred_element_type=jnp.float32)
    x0_ref[...] = (x0 + b0_ref[...]).astype(jnp.bfloat16)


def _attn_factors(f1, f2, c1, c2):
    r = jnp.exp(0.8 * (f1 - c1))
    b1v = jnp.exp(f2 - c2)
    b2v = jnp.exp(0.2 * (f2 - c2) - 0.8 * (c1 + c2))
    e = jnp.maximum(r * b1v, b2v)
    s = jnp.sum(e, axis=1, keepdims=True)
    return e.astype(jnp.bfloat16), s


def _attn1_body(f1_ref, f2_ref, c_ref, x_ref, o_ref):
    e, s = _attn_factors(f1_ref[...], f2_ref[...], c_ref[0, 0], c_ref[0, 1])
    p = jnp.dot(e, x_ref[...], preferred_element_type=jnp.float32)
    y = p / s
    y = jnp.where(y > 0.0, y, jnp.exp(y) - 1.0)
    o_ref[...] = y.astype(jnp.bfloat16)


def _attn2_body(f1_ref, f2_ref, c_ref, x_ref, w1t_ref, b1_ref, o_ref):
    e, s = _attn_factors(f1_ref[...], f2_ref[...], c_ref[0, 0], c_ref[0, 1])
    p = jnp.dot(e, x_ref[...], preferred_element_type=jnp.float32)
    y = p / s
    z = jnp.dot(y, w1t_ref[...], preferred_element_type=jnp.float32)
    z = z + b1_ref[...]
    o_ref[...] = jnp.where(z > 0.0, z, jnp.exp(z) - 1.0)


@jax.jit
def kernel(feat_data, adjs, Wf, bf, a_src, a_dest, W0, b0, W1, b1):
    del adjs
    a2 = jnp.concatenate([a_src, a_dest], axis=1)

    f, x0 = pl.pallas_call(
        _prep_body,
        grid=(NBLK,),
        in_specs=[
            pl.BlockSpec((BLK, NFEAT), lambda i: (i, 0)),
            pl.BlockSpec((NFEAT, NHID), lambda i: (0, 0)),
            pl.BlockSpec((1, NHID), lambda i: (0, 0)),
            pl.BlockSpec((NHID, 2), lambda i: (0, 0)),
            pl.BlockSpec((NFEAT, NHID), lambda i: (0, 0)),
            pl.BlockSpec((1, NHID), lambda i: (0, 0)),
        ],
        out_specs=[
            pl.BlockSpec((BLK, 2), lambda i: (i, 0)),
            pl.BlockSpec((BLK, NHID), lambda i: (i, 0)),
        ],
        out_shape=[
            jax.ShapeDtypeStruct((N, 2), jnp.float32),
            jax.ShapeDtypeStruct((N, NHID), jnp.bfloat16),
        ],
    )(feat_data, Wf.T, bf.reshape(1, NHID), a2, W0.T, b0.reshape(1, NHID))

    c = jnp.max(f, axis=0).reshape(1, 2)
    f1c = f[:, 0:1]
    f2r = f[:, 1].reshape(1, N)

    attn_specs = [
        pl.BlockSpec((BLK, 1), lambda i: (i, 0)),
        pl.BlockSpec((1, N), lambda i: (0, 0)),
        pl.BlockSpec((1, 2), lambda i: (0, 0)),
        pl.BlockSpec((N, NHID), lambda i: (0, 0)),
    ]

    y0 = pl.pallas_call(
        _attn1_body,
        grid=(NBLK,),
        in_specs=attn_specs,
        out_specs=pl.BlockSpec((BLK, NHID), lambda i: (i, 0)),
        out_shape=jax.ShapeDtypeStruct((N, NHID), jnp.bfloat16),
    )(f1c, f2r, c, x0)

    out = pl.pallas_call(
        _attn2_body,
        grid=(NBLK,),
        in_specs=attn_specs + [
            pl.BlockSpec((NHID, NHID), lambda i: (0, 0)),
            pl.BlockSpec((1, NHID), lambda i: (0, 0)),
        ],
        out_specs=pl.BlockSpec((BLK, NHID), lambda i: (i, 0)),
        out_shape=jax.ShapeDtypeStruct((N, NHID), jnp.float32),
    )(f1c, f2r, c, y0, W1.T, b1.reshape(1, NHID))

    return out


# speedup vs baseline: 2.1225x; 1.0802x over previous
import jax
import jax.numpy as jnp
from jax.experimental import pallas as pl

N = 4096
NFEAT = 128
NHID = 128
BLK = 1024
NBLK = N // BLK


def _prep_body(feat_ref, wft_ref, bf_ref, a2_ref, w0t_ref, b0_ref,
               f_ref, x0_ref):
    feat = feat_ref[...]
    h = jnp.dot(feat, wft_ref[...], preferred_element_type=jnp.float32)
    h = h + bf_ref[...]
    f_ref[...] = jnp.dot(h, a2_ref[...], preferred_element_type=jnp.float32)
    x0 = jnp.dot(feat, w0t_ref[...], preferred_element_type=jnp.float32)
    x0_ref[...] = (x0 + b0_ref[...]).astype(jnp.bfloat16)


def _attn_factors(f1, f2, c1, c2):
    r = jnp.exp(0.8 * (f1 - c1))
    b1v = jnp.exp(f2 - c2)
    b2v = jnp.exp(0.2 * (f2 - c2) - 0.8 * (c1 + c2))
    e = jnp.maximum(r * b1v, b2v)
    s = jnp.sum(e, axis=1, keepdims=True)
    return e.astype(jnp.bfloat16), s


def _attn1_body(f1_ref, f2_ref, c_ref, x_ref, o_ref):
    e, s = _attn_factors(f1_ref[...], f2_ref[...], c_ref[0, 0], c_ref[0, 1])
    p = jnp.dot(e, x_ref[...], preferred_element_type=jnp.float32)
    y = p / s
    y = jnp.where(y > 0.0, y, jnp.exp(y) - 1.0)
    o_ref[...] = y.astype(jnp.bfloat16)


def _attn2_body(f1_ref, f2_ref, c_ref, x_ref, w1t_ref, b1_ref, o_ref):
    e, s = _attn_factors(f1_ref[...], f2_ref[...], c_ref[0, 0], c_ref[0, 1])
    p = jnp.dot(e, x_ref[...], preferred_element_type=jnp.float32)
    y = p / s
    z = jnp.dot(y, w1t_ref[...], preferred_element_type=jnp.float32)
    z = z + b1_ref[...]
    o_ref[...] = jnp.where(z > 0.0, z, jnp.exp(z) - 1.0)


@jax.jit
def kernel(feat_data, adjs, Wf, bf, a_src, a_dest, W0, b0, W1, b1):
    del adjs
    a2 = jnp.concatenate([a_src, a_dest], axis=1)

    f, x0 = pl.pallas_call(
        _prep_body,
        grid=(NBLK,),
        in_specs=[
            pl.BlockSpec((BLK, NFEAT), lambda i: (i, 0)),
            pl.BlockSpec((NFEAT, NHID), lambda i: (0, 0)),
            pl.BlockSpec((1, NHID), lambda i: (0, 0)),
            pl.BlockSpec((NHID, 2), lambda i: (0, 0)),
            pl.BlockSpec((NFEAT, NHID), lambda i: (0, 0)),
            pl.BlockSpec((1, NHID), lambda i: (0, 0)),
        ],
        out_specs=[
            pl.BlockSpec((BLK, 2), lambda i: (i, 0)),
            pl.BlockSpec((BLK, NHID), lambda i: (i, 0)),
        ],
        out_shape=[
            jax.ShapeDtypeStruct((N, 2), jnp.float32),
            jax.ShapeDtypeStruct((N, NHID), jnp.bfloat16),
        ],
    )(feat_data, Wf.T, bf.reshape(1, NHID), a2, W0.T, b0.reshape(1, NHID))

    c = jnp.max(f, axis=0).reshape(1, 2)
    f1c = f[:, 0:1]
    f2r = f[:, 1].reshape(1, N)

    attn_specs = [
        pl.BlockSpec((BLK, 1), lambda i: (i, 0)),
        pl.BlockSpec((1, N), lambda i: (0, 0)),
        pl.BlockSpec((1, 2), lambda i: (0, 0)),
        pl.BlockSpec((N, NHID), lambda i: (0, 0)),
    ]

    y0 = pl.pallas_call(
        _attn1_body,
        grid=(NBLK,),
        in_specs=attn_specs,
        out_specs=pl.BlockSpec((BLK, NHID), lambda i: (i, 0)),
        out_shape=jax.ShapeDtypeStruct((N, NHID), jnp.bfloat16),
    )(f1c, f2r, c, x0)

    out = pl.pallas_call(
        _attn2_body,
        grid=(NBLK,),
        in_specs=attn_specs + [
            pl.BlockSpec((NHID, NHID), lambda i: (0, 0)),
            pl.BlockSpec((1, NHID), lambda i: (0, 0)),
        ],
        out_specs=pl.BlockSpec((BLK, NHID), lambda i: (i, 0)),
        out_shape=jax.ShapeDtypeStruct((N, NHID), jnp.float32),
    )(f1c, f2r, c, y0, W1.T, b1.reshape(1, NHID))

    return out


# speedup vs baseline: 2.1792x; 1.0267x over previous
import jax
import jax.numpy as jnp
from jax.experimental import pallas as pl

N = 4096
NFEAT = 128
NHID = 128
BLK = 2048
NBLK = N // BLK


def _prep_body(feat_ref, wft_ref, bf_ref, a2_ref, w0t_ref, b0_ref,
               f_ref, x0_ref):
    feat = feat_ref[...]
    h = jnp.dot(feat, wft_ref[...], preferred_element_type=jnp.float32)
    h = h + bf_ref[...]
    f_ref[...] = jnp.dot(h, a2_ref[...], preferred_element_type=jnp.float32)
    x0 = jnp.dot(feat, w0t_ref[...], preferred_element_type=jnp.float32)
    x0_ref[...] = (x0 + b0_ref[...]).astype(jnp.bfloat16)


def _attn_factors(f1, f2, c1, c2):
    r = jnp.exp(0.8 * (f1 - c1))
    b1v = jnp.exp(f2 - c2)
    b2v = jnp.exp(0.2 * (f2 - c2) - 0.8 * (c1 + c2))
    e = jnp.maximum(r * b1v, b2v)
    s = jnp.sum(e, axis=1, keepdims=True)
    return e.astype(jnp.bfloat16), s


def _attn1_body(f1_ref, f2_ref, c_ref, x_ref, o_ref):
    e, s = _attn_factors(f1_ref[...], f2_ref[...], c_ref[0, 0], c_ref[0, 1])
    p = jnp.dot(e, x_ref[...], preferred_element_type=jnp.float32)
    y = p / s
    y = jnp.where(y > 0.0, y, jnp.exp(y) - 1.0)
    o_ref[...] = y.astype(jnp.bfloat16)


def _attn2_body(f1_ref, f2_ref, c_ref, x_ref, w1t_ref, b1_ref, o_ref):
    e, s = _attn_factors(f1_ref[...], f2_ref[...], c_ref[0, 0], c_ref[0, 1])
    p = jnp.dot(e, x_ref[...], preferred_element_type=jnp.float32)
    y = p / s
    z = jnp.dot(y, w1t_ref[...], preferred_element_type=jnp.float32)
    z = z + b1_ref[...]
    o_ref[...] = jnp.where(z > 0.0, z, jnp.exp(z) - 1.0)


@jax.jit
def kernel(feat_data, adjs, Wf, bf, a_src, a_dest, W0, b0, W1, b1):
    del adjs
    a2 = jnp.concatenate([a_src, a_dest], axis=1)

    f, x0 = pl.pallas_call(
        _prep_body,
        grid=(NBLK,),
        in_specs=[
            pl.BlockSpec((BLK, NFEAT), lambda i: (i, 0)),
            pl.BlockSpec((NFEAT, NHID), lambda i: (0, 0)),
            pl.BlockSpec((1, NHID), lambda i: (0, 0)),
            pl.BlockSpec((NHID, 2), lambda i: (0, 0)),
            pl.BlockSpec((NFEAT, NHID), lambda i: (0, 0)),
            pl.BlockSpec((1, NHID), lambda i: (0, 0)),
        ],
        out_specs=[
            pl.BlockSpec((BLK, 2), lambda i: (i, 0)),
            pl.BlockSpec((BLK, NHID), lambda i: (i, 0)),
        ],
        out_shape=[
            jax.ShapeDtypeStruct((N, 2), jnp.float32),
            jax.ShapeDtypeStruct((N, NHID), jnp.bfloat16),
        ],
    )(feat_data, Wf.T, bf.reshape(1, NHID), a2, W0.T, b0.reshape(1, NHID))

    c = jnp.max(f, axis=0).reshape(1, 2)
    f1c = f[:, 0:1]
    f2r = f[:, 1].reshape(1, N)

    attn_specs = [
        pl.BlockSpec((BLK, 1), lambda i: (i, 0)),
        pl.BlockSpec((1, N), lambda i: (0, 0)),
        pl.BlockSpec((1, 2), lambda i: (0, 0)),
        pl.BlockSpec((N, NHID), lambda i: (0, 0)),
    ]

    y0 = pl.pallas_call(
        _attn1_body,
        grid=(NBLK,),
        in_specs=attn_specs,
        out_specs=pl.BlockSpec((BLK, NHID), lambda i: (i, 0)),
        out_shape=jax.ShapeDtypeStruct((N, NHID), jnp.bfloat16),
    )(f1c, f2r, c, x0)

    out = pl.pallas_call(
        _attn2_body,
        grid=(NBLK,),
        in_specs=attn_specs + [
            pl.BlockSpec((NHID, NHID), lambda i: (0, 0)),
            pl.BlockSpec((1, NHID), lambda i: (0, 0)),
        ],
        out_specs=pl.BlockSpec((BLK, NHID), lambda i: (i, 0)),
        out_shape=jax.ShapeDtypeStruct((N, NHID), jnp.float32),
    )(f1c, f2r, c, y0, W1.T, b1.reshape(1, NHID))

    return out


# speedup vs baseline: 2.7158x; 1.2462x over previous
import jax
import jax.numpy as jnp
from jax.experimental import pallas as pl

N = 4096
NFEAT = 128
NHID = 128
XW = 2 * NHID
BLK = 2048
NBLK = N // BLK


def _prep_body(feat_ref, wft_ref, bf_ref, a2_ref, w0t_ref, b0_ref,
               f_ref, x0_ref):
    feat = feat_ref[...]
    h = jnp.dot(feat, wft_ref[...], preferred_element_type=jnp.float32)
    h = h + bf_ref[...]
    f_ref[...] = jnp.dot(h, a2_ref[...], preferred_element_type=jnp.float32)
    x0 = jnp.dot(feat, w0t_ref[...], preferred_element_type=jnp.float32)
    x0 = (x0 + b0_ref[...]).astype(jnp.bfloat16)
    ones = jnp.ones((x0.shape[0], NHID), jnp.bfloat16)
    x0_ref[...] = jnp.concatenate([x0, ones], axis=1)


def _attn_e(f1, f2, c1, c2):
    r = jnp.exp(0.8 * (f1 - c1)).astype(jnp.bfloat16)
    b1v = jnp.exp(f2 - c2).astype(jnp.bfloat16)
    b2v = jnp.exp(0.2 * (f2 - c2) - 0.8 * (c1 + c2)).astype(jnp.bfloat16)
    return jnp.maximum(r * b1v, b2v)


def _attn1_body(f1_ref, f2_ref, c_ref, x_ref, o_ref):
    e = _attn_e(f1_ref[...], f2_ref[...], c_ref[0, 0], c_ref[0, 1])
    p = jnp.dot(e, x_ref[...], preferred_element_type=jnp.float32)
    y = p[:, :NHID] / p[:, NHID:NHID + 1]
    y = jnp.where(y > 0.0, y, jnp.exp(y) - 1.0)
    ones = jnp.ones((y.shape[0], NHID), jnp.bfloat16)
    o_ref[...] = jnp.concatenate([y.astype(jnp.bfloat16), ones], axis=1)


def _attn2_body(f1_ref, f2_ref, c_ref, x_ref, w1t_ref, b1_ref, o_ref):
    e = _attn_e(f1_ref[...], f2_ref[...], c_ref[0, 0], c_ref[0, 1])
    p = jnp.dot(e, x_ref[...], preferred_element_type=jnp.float32)
    y = p[:, :NHID] / p[:, NHID:NHID + 1]
    z = jnp.dot(y, w1t_ref[...], preferred_element_type=jnp.float32)
    z = z + b1_ref[...]
    o_ref[...] = jnp.where(z > 0.0, z, jnp.exp(z) - 1.0)


@jax.jit
def kernel(feat_data, adjs, Wf, bf, a_src, a_dest, W0, b0, W1, b1):
    del adjs
    a2 = jnp.concatenate([a_src, a_dest], axis=1)

    f, x0 = pl.pallas_call(
        _prep_body,
        grid=(NBLK,),
        in_specs=[
            pl.BlockSpec((BLK, NFEAT), lambda i: (i, 0)),
            pl.BlockSpec((NFEAT, NHID), lambda i: (0, 0)),
            pl.BlockSpec((1, NHID), lambda i: (0, 0)),
            pl.BlockSpec((NHID, 2), lambda i: (0, 0)),
            pl.BlockSpec((NFEAT, NHID), lambda i: (0, 0)),
            pl.BlockSpec((1, NHID), lambda i: (0, 0)),
        ],
        out_specs=[
            pl.BlockSpec((BLK, 2), lambda i: (i, 0)),
            pl.BlockSpec((BLK, XW), lambda i: (i, 0)),
        ],
        out_shape=[
            jax.ShapeDtypeStruct((N, 2), jnp.float32),
            jax.ShapeDtypeStruct((N, XW), jnp.bfloat16),
        ],
    )(feat_data, Wf.T, bf.reshape(1, NHID), a2, W0.T, b0.reshape(1, NHID))

    c = jnp.max(f, axis=0).reshape(1, 2)
    f1c = f[:, 0:1]
    f2r = f[:, 1].reshape(1, N)

    attn_specs = [
        pl.BlockSpec((BLK, 1), lambda i: (i, 0)),
        pl.BlockSpec((1, N), lambda i: (0, 0)),
        pl.BlockSpec((1, 2), lambda i: (0, 0)),
        pl.BlockSpec((N, XW), lambda i: (0, 0)),
    ]

    y0 = pl.pallas_call(
        _attn1_body,
        grid=(NBLK,),
        in_specs=attn_specs,
        out_specs=pl.BlockSpec((BLK, XW), lambda i: (i, 0)),
        out_shape=jax.ShapeDtypeStruct((N, XW), jnp.bfloat16),
    )(f1c, f2r, c, x0)

    out = pl.pallas_call(
        _attn2_body,
        grid=(NBLK,),
        in_specs=attn_specs + [
            pl.BlockSpec((NHID, NHID), lambda i: (0, 0)),
            pl.BlockSpec((1, NHID), lambda i: (0, 0)),
        ],
        out_specs=pl.BlockSpec((BLK, NHID), lambda i: (i, 0)),
        out_shape=jax.ShapeDtypeStruct((N, NHID), jnp.float32),
    )(f1c, f2r, c, y0, W1.T, b1.reshape(1, NHID))

    return out


# speedup vs baseline: 3.0744x; 1.1320x over previous
import jax
import jax.numpy as jnp
from jax.experimental import pallas as pl

N = 4096
NFEAT = 128
NHID = 128
XW = 2 * NHID
BLK = 2048
NBLK = N // BLK


def _prep_body(feat_ref, wft_ref, bf_ref, a2_ref, w0t_ref, b0_ref,
               f1_ref, f2r_ref, c_ref, x0_ref):
    feat = feat_ref[...]
    h = jnp.dot(feat, wft_ref[...], preferred_element_type=jnp.float32)
    h = h + bf_ref[...]
    f = jnp.dot(h, a2_ref[...], preferred_element_type=jnp.float32)
    f1_ref[...] = f[:, 0:1]
    f2r_ref[...] = f[:, 1:2].T
    c_ref[...] = jnp.max(f, axis=0, keepdims=True)
    x0 = jnp.dot(feat, w0t_ref[...], preferred_element_type=jnp.float32)
    x0 = (x0 + b0_ref[...]).astype(jnp.bfloat16)
    ones = jnp.ones((N, NHID), jnp.bfloat16)
    x0_ref[...] = jnp.concatenate([x0, ones], axis=1)


def _attn_e(f1, f2, c1, c2):
    r = jnp.exp(0.8 * (f1 - c1)).astype(jnp.bfloat16)
    b1v = jnp.exp(f2 - c2).astype(jnp.bfloat16)
    b2v = jnp.exp(0.2 * (f2 - c2) - 0.8 * (c1 + c2)).astype(jnp.bfloat16)
    return jnp.maximum(r * b1v, b2v)


def _attn1_body(f1_ref, f2_ref, c_ref, x_ref, o_ref):
    e = _attn_e(f1_ref[...], f2_ref[...], c_ref[0, 0], c_ref[0, 1])
    p = jnp.dot(e, x_ref[...], preferred_element_type=jnp.float32)
    y = p[:, :NHID] / p[:, NHID:NHID + 1]
    y = jnp.where(y > 0.0, y, jnp.exp(y) - 1.0)
    ones = jnp.ones((y.shape[0], NHID), jnp.bfloat16)
    o_ref[...] = jnp.concatenate([y.astype(jnp.bfloat16), ones], axis=1)


def _attn2_body(f1_ref, f2_ref, c_ref, x_ref, w1t_ref, b1_ref, o_ref):
    e = _attn_e(f1_ref[...], f2_ref[...], c_ref[0, 0], c_ref[0, 1])
    p = jnp.dot(e, x_ref[...], preferred_element_type=jnp.float32)
    y = p[:, :NHID] / p[:, NHID:NHID + 1]
    z = jnp.dot(y, w1t_ref[...], preferred_element_type=jnp.float32)
    z = z + b1_ref[...]
    o_ref[...] = jnp.where(z > 0.0, z, jnp.exp(z) - 1.0)


@jax.jit
def kernel(feat_data, adjs, Wf, bf, a_src, a_dest, W0, b0, W1, b1):
    del adjs
    a2 = jnp.concatenate([a_src, a_dest], axis=1)

    f1c, f2r, c, x0 = pl.pallas_call(
        _prep_body,
        out_shape=[
            jax.ShapeDtypeStruct((N, 1), jnp.float32),
            jax.ShapeDtypeStruct((1, N), jnp.float32),
            jax.ShapeDtypeStruct((1, 2), jnp.float32),
            jax.ShapeDtypeStruct((N, XW), jnp.bfloat16),
        ],
    )(feat_data, Wf.T, bf.reshape(1, NHID), a2, W0.T, b0.reshape(1, NHID))

    attn_specs = [
        pl.BlockSpec((BLK, 1), lambda i: (i, 0)),
        pl.BlockSpec((1, N), lambda i: (0, 0)),
        pl.BlockSpec((1, 2), lambda i: (0, 0)),
        pl.BlockSpec((N, XW), lambda i: (0, 0)),
    ]

    y0 = pl.pallas_call(
        _attn1_body,
        grid=(NBLK,),
        in_specs=attn_specs,
        out_specs=pl.BlockSpec((BLK, XW), lambda i: (i, 0)),
        out_shape=jax.ShapeDtypeStruct((N, XW), jnp.bfloat16),
    )(f1c, f2r, c, x0)

    out = pl.pallas_call(
        _attn2_body,
        grid=(NBLK,),
        in_specs=attn_specs + [
            pl.BlockSpec((NHID, NHID), lambda i: (0, 0)),
            pl.BlockSpec((1, NHID), lambda i: (0, 0)),
        ],
        out_specs=pl.BlockSpec((BLK, NHID), lambda i: (i, 0)),
        out_shape=jax.ShapeDtypeStruct((N, NHID), jnp.float32),
    )(f1c, f2r, c, y0, W1.T, b1.reshape(1, NHID))

    return out
